# baseline (speedup 1.0000x reference)
import functools

import jax
import jax.numpy as jnp
from jax import lax
from jax.experimental import pallas as pl
from jax.experimental.pallas import tpu as pltpu
from jax.experimental.pallas import tpu_sc as plsc

BATCH = 16384
EMBED_DIM = 64
CHUNK = 128
LANES = 16
NENT = 1000000
BLK = 128
LAST_FULL_BLK = NENT // BLK - 1
RING = 256

_info = plsc.get_sparse_core_info()
_NC, _NS = _info.num_cores, _info.num_subcores
_NW = _NC * _NS
_B_PER_W = BATCH // _NW
_N_CHUNKS = _B_PER_W // CHUNK
_HITS_W = 2 * BATCH // _NW


def _make_entity_kernel():
    out_t = tuple(
        jax.ShapeDtypeStruct((BATCH, EMBED_DIM), jnp.float32) for _ in range(2)
    )
    scratch = (
        [pltpu.VMEM((_HITS_W,), jnp.int32),
         pltpu.VMEM((_HITS_W,), jnp.int32),
         pltpu.VMEM((EMBED_DIM, BLK), jnp.float32),
         pltpu.VMEM((EMBED_DIM, BLK), jnp.float32),
         pltpu.VMEM((EMBED_DIM, BLK), jnp.float32),
         pltpu.VMEM((RING, EMBED_DIM), jnp.float32)]
        + [pltpu.SemaphoreType.DMA,
           pltpu.SemaphoreType.DMA,
           pltpu.SemaphoreType.DMA,
           pltpu.SemaphoreType.DMA]
    )

    @functools.partial(
        pl.kernel,
        mesh=plsc.VectorSubcoreMesh(core_axis_name="c", subcore_axis_name="s"),
        out_type=out_t,
        compiler_params=pltpu.CompilerParams(needs_layout_passes=False),
        scratch_types=scratch,
    )
    def k(skey_h, spos_h, ent_t, tail_t, out0, out2,
          sv_ref, sp_ref, buf0, buf1, tailb, ring,
          isem, bsem0, bsem1, wsem):
        wid = lax.axis_index("s") * _NC + lax.axis_index("c")
        seg = wid * _HITS_W

        cps = [
            pltpu.async_copy(skey_h.at[pl.ds(seg, _HITS_W)], sv_ref, isem),
            pltpu.async_copy(spos_h.at[pl.ds(seg, _HITS_W)], sp_ref, isem),
            pltpu.async_copy(tail_t, tailb, isem),
        ]
        for cp in cps:
            cp.wait()

        blo = lax.shift_right_logical(sv_ref[pl.ds(0, LANES)][0], 7)
        bhi = lax.shift_right_logical(
            sv_ref[pl.ds(_HITS_W - LANES, LANES)][LANES - 1], 7
        )
        bhi_eff = jnp.minimum(bhi, LAST_FULL_BLK)
        lane = lax.iota(jnp.int32, LANES)

        def fetch(b, buf, sem):
            return pltpu.async_copy(
                ent_t.at[:, pl.ds(b * BLK, BLK)], buf, sem
            )

        def drain(sem, buf):
            pltpu.make_async_copy(
                ent_t.at[:, pl.ds(0, BLK)], buf, sem
            ).wait()

        def process_block(b, buf, carry_in, r_off=0):
            def group(c):
                ivec = jnp.minimum(
                    jnp.full((LANES,), c, jnp.int32) + lane, _HITS_W - 1
                )
                sv = plsc.load_gather(sv_ref, [ivec])
                po = plsc.load_gather(sp_ref, [ivec])
                valid = (jnp.full((LANES,), c, jnp.int32) + lane) < _HITS_W
                m = jnp.logical_and(lax.shift_right_logical(sv, 7) == b, valid)
                n = plsc.all_reduce_population_count(m)[0]
                return sv, po, m, n

            def cond(carry):
                c, dr, sv, po, m, n = carry
                return n > 0

            def body(carry):
                c, dr, sv, po, m, n = carry
                r = lax.bitwise_and(sv, BLK - 1) + r_off
                slot = lax.bitwise_and(
                    jnp.full((LANES,), c, jnp.int32) + lane, RING - 1
                )

                @pl.when(dr < c + LANES - RING)
                def _():
                    pltpu.make_async_copy(
                        ring.at[pl.ds(0, LANES)],
                        out0.at[pl.ds(0, LANES)],
                        wsem,
                    ).wait()

                dr = lax.select(dr < c + LANES - RING, dr + LANES, dr)

                for d in range(EMBED_DIM):
                    v = plsc.load_gather(
                        buf, [jnp.full((LANES,), d, jnp.int32), r], mask=m
                    )
                    plsc.store_scatter(
                        ring, [slot, jnp.full((LANES,), d, jnp.int32)], v,
                        mask=m,
                    )
                for j in range(LANES):
                    pj = po[j]

                    @pl.when(jnp.logical_and(j < n, pj < BATCH))
                    def _():
                        pltpu.async_copy(
                            ring.at[lax.bitwise_and(c + j, RING - 1)],
                            out0.at[pj], wsem,
                        )

                    @pl.when(jnp.logical_and(j < n, pj >= BATCH))
                    def _():
                        pltpu.async_copy(
                            ring.at[lax.bitwise_and(c + j, RING - 1)],
                            out2.at[lax.bitwise_and(pj, BATCH - 1)], wsem,
                        )

                c2 = c + n
                nsv, npo, nm, nn = group(c2)
                nn = lax.select(n == LANES, nn, jnp.int32(0))
                return c2, dr, nsv, npo, nm, nn

            cursor, dr = carry_in
            sv0, po0, m0, n0 = group(cursor)
            c_out, dr_out, _, _, _, _ = lax.while_loop(
                cond, body, (cursor, dr, sv0, po0, m0, n0)
            )
            return c_out, dr_out

        @pl.when(blo <= bhi_eff)
        def _():
            fetch(blo, buf0, bsem0)
            drain(bsem0, buf0)

        npairs = lax.select(
            blo <= bhi_eff,
            lax.shift_right_logical(bhi_eff - blo + 2, 1),
            jnp.int32(0),
        )

        def pair_body(g, carry):
            b0 = blo + 2 * g

            @pl.when(b0 + 1 <= bhi_eff)
            def _():
                fetch(b0 + 1, buf1, bsem1)

            carry = process_block(b0, buf0, carry)

            @pl.when(b0 + 2 <= bhi_eff)
            def _():
                fetch(b0 + 2, buf0, bsem0)

            @pl.when(b0 + 1 <= bhi_eff)
            def _():
                drain(bsem1, buf1)

            b1 = lax.select(b0 + 1 <= bhi_eff, b0 + 1, jnp.int32(-1))
            carry = process_block(b1, buf1, carry)

            @pl.when(b0 + 2 <= bhi_eff)
            def _():
                drain(bsem0, buf0)

            return carry

        carry = lax.fori_loop(0, npairs, pair_body,
                              (jnp.int32(0), jnp.int32(0)))
        _, dr = process_block(jnp.int32(LAST_FULL_BLK + 1), tailb, carry,
                              r_off=EMBED_DIM)
        ndrains = lax.shift_right_logical(_HITS_W - dr, 4)

        def final_drain(i, x):
            pltpu.make_async_copy(
                ring.at[pl.ds(0, LANES)], out0.at[pl.ds(0, LANES)], wsem
            ).wait()
            return x

        lax.fori_loop(0, ndrains, final_drain, 0)

    return k


def _make_small_kernel():
    out_t = tuple(
        jax.ShapeDtypeStruct((BATCH, EMBED_DIM), jnp.float32) for _ in range(2)
    )
    scratch = (
        [pltpu.VMEM((_N_CHUNKS, CHUNK), jnp.int32) for _ in range(2)]
        + [pltpu.VMEM((CHUNK, EMBED_DIM), jnp.float32) for _ in range(2)]
        + [pltpu.SemaphoreType.DMA,
           pltpu.SemaphoreType.DMA,
           pltpu.SemaphoreType.DMA]
    )

    @functools.partial(
        pl.kernel,
        mesh=plsc.VectorSubcoreMesh(core_axis_name="c", subcore_axis_name="s"),
        out_type=out_t,
        compiler_params=pltpu.CompilerParams(use_tc_tiling_on_sc=False),
        scratch_types=scratch,
    )
    def k(rel_h, ts_h, rel_t, ts_t, out1, out3, ir, its, buf0, buf1,
          isem, gsem, wsem):
        wid = lax.axis_index("s") * _NC + lax.axis_index("c")
        row_base = wid * _B_PER_W
        chunk_base = wid * _N_CHUNKS

        for cp in [
            pltpu.async_copy(src.at[pl.ds(chunk_base, _N_CHUNKS)], dst, isem)
            for src, dst in ((rel_h, ir), (ts_h, its))
        ]:
            cp.wait()

        bufs = (buf0, buf1)
        pending = [None, None]
        tasks = [
            (idx.at[c], table, out.at[pl.ds(row_base + c * CHUNK, CHUNK)])
            for idx, table, out in ((ir, rel_t, out1), (its, ts_t, out3))
            for c in range(_N_CHUNKS)
        ]
        for t, (idx, table, dst) in enumerate(tasks):
            s = t % 2
            if pending[s] is not None:
                pending[s].wait()
            pltpu.async_copy(table.at[idx], bufs[s], gsem).wait()
            pending[s] = pltpu.async_copy(bufs[s], dst, wsem)
        for s in range(2):
            if pending[s] is not None:
                pending[s].wait()

    return k


_ent_lookup = _make_entity_kernel()
_small_lookup = _make_small_kernel()


def kernel(head, relation, tail, timestamp, entity_table, relation_table, timestamp_table):
    idx2 = lambda a: a.reshape(BATCH // CHUNK, CHUNK)
    keys = jnp.concatenate([head, tail])
    vals = jnp.arange(2 * BATCH, dtype=jnp.int32)
    skeys, spos = lax.sort_key_val(keys, vals)
    tail_rows = lax.slice(entity_table, (NENT - BLK, 0), (NENT, EMBED_DIM))
    out0, out2 = _ent_lookup(skeys, spos, entity_table.T, tail_rows.T)
    out1, out3 = _small_lookup(
        idx2(relation), idx2(timestamp), relation_table, timestamp_table
    )
    return (out0, out1, out2, out3)


# speedup vs baseline: 1.0605x; 1.0605x over previous
import functools

import jax
import jax.numpy as jnp
from jax import lax
from jax.experimental import pallas as pl
from jax.experimental.pallas import tpu as pltpu
from jax.experimental.pallas import tpu_sc as plsc

BATCH = 16384
EMBED_DIM = 64
CHUNK = 128
LANES = 16
NENT = 1000000
BLK = 128
LAST_FULL_BLK = NENT // BLK - 1
RING = 256

_info = plsc.get_sparse_core_info()
_NC, _NS = _info.num_cores, _info.num_subcores
_NW = _NC * _NS
_B_PER_W = BATCH // _NW
_N_CHUNKS = _B_PER_W // CHUNK
_HITS_W = 2 * BATCH // _NW


def _make_entity_kernel():
    out_t = tuple(
        jax.ShapeDtypeStruct((BATCH, EMBED_DIM), jnp.float32) for _ in range(2)
    )
    scratch = (
        [pltpu.VMEM((_HITS_W,), jnp.int32),
         pltpu.VMEM((_HITS_W,), jnp.int32),
         pltpu.VMEM((EMBED_DIM, BLK), jnp.float32),
         pltpu.VMEM((EMBED_DIM, BLK), jnp.float32),
         pltpu.VMEM((EMBED_DIM, BLK), jnp.float32),
         pltpu.VMEM((RING, EMBED_DIM), jnp.float32)]
        + [pltpu.SemaphoreType.DMA,
           pltpu.SemaphoreType.DMA,
           pltpu.SemaphoreType.DMA,
           pltpu.SemaphoreType.DMA]
    )

    @functools.partial(
        pl.kernel,
        mesh=plsc.VectorSubcoreMesh(core_axis_name="c", subcore_axis_name="s"),
        out_type=out_t,
        compiler_params=pltpu.CompilerParams(needs_layout_passes=False),
        scratch_types=scratch,
    )
    def k(skey_h, spos_h, ent_t, tail_t, out0, out2,
          sv_ref, sp_ref, buf0, buf1, tailb, ring,
          isem, bsem0, bsem1, wsem):
        wid = lax.axis_index("s") * _NC + lax.axis_index("c")
        seg = wid * _HITS_W

        cps = [
            pltpu.async_copy(skey_h.at[pl.ds(seg, _HITS_W)], sv_ref, isem),
            pltpu.async_copy(spos_h.at[pl.ds(seg, _HITS_W)], sp_ref, isem),
            pltpu.async_copy(tail_t, tailb, isem),
        ]
        for cp in cps:
            cp.wait()

        blo = lax.shift_right_logical(sv_ref[pl.ds(0, LANES)][0], 7)
        bhi = lax.shift_right_logical(
            sv_ref[pl.ds(_HITS_W - LANES, LANES)][LANES - 1], 7
        )
        bhi_eff = jnp.minimum(bhi, LAST_FULL_BLK)
        lane = lax.iota(jnp.int32, LANES)

        def fetch(b, buf, sem):
            return pltpu.async_copy(
                ent_t.at[:, pl.ds(b * BLK, BLK)], buf, sem
            )

        def drain(sem, buf):
            pltpu.make_async_copy(
                ent_t.at[:, pl.ds(0, BLK)], buf, sem
            ).wait()

        def process_block(b, buf, cursor, r_off=0):
            def hit(c):
                cm = jnp.minimum(c, _HITS_W - 1)
                sv = plsc.load_gather(sv_ref, [jnp.full((LANES,), cm, jnp.int32)])[0]
                po = plsc.load_gather(sp_ref, [jnp.full((LANES,), cm, jnp.int32)])[0]
                return sv, po

            def cond(carry):
                c, sv, po = carry
                return jnp.logical_and(
                    c < _HITS_W, lax.shift_right_logical(sv, 7) == b
                )

            def body(carry):
                c, sv, po = carry
                r = lax.bitwise_and(sv, BLK - 1) + r_off
                slot = lax.bitwise_and(c, RING - 1)

                @pl.when(jnp.logical_and(c >= RING,
                                         lax.bitwise_and(c, 15) == 0))
                def _():
                    pltpu.make_async_copy(
                        ring.at[pl.ds(0, LANES)],
                        out0.at[pl.ds(0, LANES)],
                        wsem,
                    ).wait()

                for q in range(EMBED_DIM // LANES):
                    v = plsc.load_gather(
                        buf, [lane + q * LANES, jnp.full((LANES,), r, jnp.int32)]
                    )
                    ring[slot, pl.ds(q * LANES, LANES)] = v
                p = lax.bitwise_and(po, BATCH - 1)
                row_src = ring.at[slot]

                @pl.when(po < BATCH)
                def _():
                    pltpu.async_copy(row_src, out0.at[p], wsem)

                @pl.when(po >= BATCH)
                def _():
                    pltpu.async_copy(row_src, out2.at[p], wsem)

                nsv, npo = hit(c + 1)
                return c + 1, nsv, npo

            sv0, po0 = hit(cursor)
            c_out, _, _ = lax.while_loop(cond, body, (cursor, sv0, po0))
            return c_out

        @pl.when(blo <= bhi_eff)
        def _():
            fetch(blo, buf0, bsem0)
            drain(bsem0, buf0)

        npairs = lax.select(
            blo <= bhi_eff,
            lax.shift_right_logical(bhi_eff - blo + 2, 1),
            jnp.int32(0),
        )

        def pair_body(g, cursor):
            b0 = blo + 2 * g

            @pl.when(b0 + 1 <= bhi_eff)
            def _():
                fetch(b0 + 1, buf1, bsem1)

            cursor = process_block(b0, buf0, cursor)

            @pl.when(b0 + 2 <= bhi_eff)
            def _():
                fetch(b0 + 2, buf0, bsem0)

            @pl.when(b0 + 1 <= bhi_eff)
            def _():
                drain(bsem1, buf1)

            b1 = lax.select(b0 + 1 <= bhi_eff, b0 + 1, jnp.int32(-1))
            cursor = process_block(b1, buf1, cursor)

            @pl.when(b0 + 2 <= bhi_eff)
            def _():
                drain(bsem0, buf0)

            return cursor

        cursor = lax.fori_loop(0, npairs, pair_body, jnp.int32(0))
        cursor = process_block(jnp.int32(LAST_FULL_BLK + 1), tailb, cursor,
                               r_off=EMBED_DIM)
        for _ in range(RING):
            pltpu.make_async_copy(ring.at[0], out0.at[0], wsem).wait()

    return k


def _make_small_kernel():
    out_t = tuple(
        jax.ShapeDtypeStruct((BATCH, EMBED_DIM), jnp.float32) for _ in range(2)
    )
    scratch = (
        [pltpu.VMEM((_N_CHUNKS, CHUNK), jnp.int32) for _ in range(2)]
        + [pltpu.VMEM((CHUNK, EMBED_DIM), jnp.float32) for _ in range(2)]
        + [pltpu.SemaphoreType.DMA,
           pltpu.SemaphoreType.DMA,
           pltpu.SemaphoreType.DMA]
    )

    @functools.partial(
        pl.kernel,
        mesh=plsc.VectorSubcoreMesh(core_axis_name="c", subcore_axis_name="s"),
        out_type=out_t,
        compiler_params=pltpu.CompilerParams(use_tc_tiling_on_sc=False),
        scratch_types=scratch,
    )
    def k(rel_h, ts_h, rel_t, ts_t, out1, out3, ir, its, buf0, buf1,
          isem, gsem, wsem):
        wid = lax.axis_index("s") * _NC + lax.axis_index("c")
        row_base = wid * _B_PER_W
        chunk_base = wid * _N_CHUNKS

        for cp in [
            pltpu.async_copy(src.at[pl.ds(chunk_base, _N_CHUNKS)], dst, isem)
            for src, dst in ((rel_h, ir), (ts_h, its))
        ]:
            cp.wait()

        bufs = (buf0, buf1)
        pending = [None, None]
        tasks = [
            (idx.at[c], table, out.at[pl.ds(row_base + c * CHUNK, CHUNK)])
            for idx, table, out in ((ir, rel_t, out1), (its, ts_t, out3))
            for c in range(_N_CHUNKS)
        ]
        for t, (idx, table, dst) in enumerate(tasks):
            s = t % 2
            if pending[s] is not None:
                pending[s].wait()
            pltpu.async_copy(table.at[idx], bufs[s], gsem).wait()
            pending[s] = pltpu.async_copy(bufs[s], dst, wsem)
        for s in range(2):
            if pending[s] is not None:
                pending[s].wait()

    return k


_ent_lookup = _make_entity_kernel()
_small_lookup = _make_small_kernel()


def kernel(head, relation, tail, timestamp, entity_table, relation_table, timestamp_table):
    idx2 = lambda a: a.reshape(BATCH // CHUNK, CHUNK)
    keys = jnp.concatenate([head, tail])
    vals = jnp.arange(2 * BATCH, dtype=jnp.int32)
    skeys, spos = lax.sort_key_val(keys, vals)
    tail_rows = lax.slice(entity_table, (NENT - BLK, 0), (NENT, EMBED_DIM))
    out0, out2 = _ent_lookup(skeys, spos, entity_table.T, tail_rows.T)
    out1, out3 = _small_lookup(
        idx2(relation), idx2(timestamp), relation_table, timestamp_table
    )
    return (out0, out1, out2, out3)
